# Initial kernel scaffold; baseline (speedup 1.0000x reference)
import jax
import jax.numpy as jnp
from jax.experimental import pallas as pl


def kernel(x, weight, indices):
    raise NotImplementedError("write your pallas kernel here")


# speedup vs baseline: 1.7008x; 1.7008x over previous
import functools

import jax
import jax.numpy as jnp
from jax import lax
from jax.experimental import pallas as pl
from jax.experimental.pallas import tpu as pltpu
from jax.experimental.pallas import tpu_sc as plsc

_C16 = (
    (0.0, 0.0, 0.0, 0.0),
    (0.0, 0.0, 0.0, 1.0),
    (0.0, 1.0, 0.0, -1.0),
    (0.0, 1.0, 0.0, 0.0),
    (0.0, 0.0, 1.0, -1.0),
    (0.0, 0.0, 1.0, 0.0),
    (0.0, 1.0, 1.0, -2.0),
    (0.0, 1.0, 1.0, -1.0),
    (1.0, -1.0, -1.0, 1.0),
    (1.0, -1.0, -1.0, 2.0),
    (1.0, 0.0, -1.0, 0.0),
    (1.0, 0.0, -1.0, 1.0),
    (1.0, -1.0, 0.0, 0.0),
    (1.0, -1.0, 0.0, 1.0),
    (1.0, 0.0, 0.0, -1.0),
    (1.0, 0.0, 0.0, 0.0),
)

_NC, _NS = 2, 16
_NW = _NC * _NS
_LANES = 16


def _coef_body(w_ref, c4t_ref, o_ref):
    w = w_ref[...]
    m = jnp.max(w, axis=-1, keepdims=True)
    e = jnp.exp(w - m)
    p = e / jnp.sum(e, axis=-1, keepdims=True)
    o_ref[...] = lax.dot_general(
        c4t_ref[...], p, (((1,), (1,)), ((), ())),
        preferred_element_type=jnp.float32)


def _compute_coef(weight):
    out_dim = weight.shape[0]
    c4t = jnp.array(_C16, dtype=jnp.float32).T
    return pl.pallas_call(
        _coef_body,
        out_shape=jax.ShapeDtypeStruct((4, out_dim), jnp.float32),
    )(weight, c4t)


def _make_sc_kernel(batch, in_dim, out_dim):
    rows_per_tile = batch // _NW
    pass_rows = 4
    npass = rows_per_tile // pass_rows
    w = 2048
    nchunk = out_dim // w

    mesh = plsc.VectorSubcoreMesh(
        core_axis_name="c", subcore_axis_name="s",
        num_cores=_NC, num_subcores=_NS)

    @functools.partial(
        pl.kernel,
        out_type=jax.ShapeDtypeStruct((batch * out_dim,), jnp.float32),
        mesh=mesh,
        compiler_params=pltpu.CompilerParams(needs_layout_passes=False),
        scratch_types=[
            pltpu.VMEM((pass_rows * in_dim,), jnp.float32),
            pltpu.VMEM((2 * w,), jnp.int32),
            pltpu.VMEM((4 * w,), jnp.float32),
            pltpu.VMEM((pass_rows * w,), jnp.float32),
        ],
    )
    def sc_kernel(x_hbm, idx_hbm, coef_hbm, out_hbm, xbuf, ibuf, cbuf, obuf):
        wid = lax.axis_index("s") * _NC + lax.axis_index("c")
        rbase = wid * rows_per_tile

        for p in range(npass):
            prow = rbase + p * pass_rows
            for r in range(pass_rows):
                pltpu.sync_copy(
                    x_hbm.at[pl.ds((prow + r) * in_dim, in_dim)],
                    xbuf.at[pl.ds(r * in_dim, in_dim)])
            for c in range(nchunk):
                pltpu.sync_copy(idx_hbm.at[pl.ds(c * w, w)],
                                ibuf.at[pl.ds(0, w)])
                pltpu.sync_copy(idx_hbm.at[pl.ds(out_dim + c * w, w)],
                                ibuf.at[pl.ds(w, w)])
                for k in range(4):
                    pltpu.sync_copy(coef_hbm.at[pl.ds(k * out_dim + c * w, w)],
                                    cbuf.at[pl.ds(k * w, w)])

                def jbody(j, _):
                    o = j * _LANES
                    i0 = ibuf[pl.ds(o, _LANES)]
                    i1 = ibuf[pl.ds(w + o, _LANES)]
                    c0 = cbuf[pl.ds(o, _LANES)]
                    ca = cbuf[pl.ds(w + o, _LANES)]
                    cb = cbuf[pl.ds(2 * w + o, _LANES)]
                    cab = cbuf[pl.ds(3 * w + o, _LANES)]
                    for r in range(pass_rows):
                        a = plsc.load_gather(xbuf, [i0 + r * in_dim])
                        b = plsc.load_gather(xbuf, [i1 + r * in_dim])
                        obuf[pl.ds(r * w + o, _LANES)] = (
                            c0 + a * ca + b * cb + (a * b) * cab)
                    return 0

                lax.fori_loop(0, w // _LANES, jbody, 0)
                for r in range(pass_rows):
                    pltpu.sync_copy(
                        obuf.at[pl.ds(r * w, w)],
                        out_hbm.at[pl.ds((prow + r) * out_dim + c * w, w)])

    return sc_kernel


def kernel(x, weight, indices):
    batch, in_dim = x.shape
    out_dim = weight.shape[0]
    coef = _compute_coef(weight)
    idx = indices.astype(jnp.int32).reshape(-1)
    sc = _make_sc_kernel(batch, in_dim, out_dim)
    out = sc(x.reshape(-1), idx, coef.reshape(-1))
    return out.reshape(batch, out_dim)


# speedup vs baseline: 2.6403x; 1.5524x over previous
import functools

import jax
import jax.numpy as jnp
from jax import lax
from jax.experimental import pallas as pl
from jax.experimental.pallas import tpu as pltpu
from jax.experimental.pallas import tpu_sc as plsc

_C16 = (
    (0.0, 0.0, 0.0, 0.0),
    (0.0, 0.0, 0.0, 1.0),
    (0.0, 1.0, 0.0, -1.0),
    (0.0, 1.0, 0.0, 0.0),
    (0.0, 0.0, 1.0, -1.0),
    (0.0, 0.0, 1.0, 0.0),
    (0.0, 1.0, 1.0, -2.0),
    (0.0, 1.0, 1.0, -1.0),
    (1.0, -1.0, -1.0, 1.0),
    (1.0, -1.0, -1.0, 2.0),
    (1.0, 0.0, -1.0, 0.0),
    (1.0, 0.0, -1.0, 1.0),
    (1.0, -1.0, 0.0, 0.0),
    (1.0, -1.0, 0.0, 1.0),
    (1.0, 0.0, 0.0, -1.0),
    (1.0, 0.0, 0.0, 0.0),
)

_NC, _NS = 2, 16
_NW = _NC * _NS
_LANES = 16


def _coef_body(w_ref, c4t_ref, o_ref):
    w = w_ref[...]
    m = jnp.max(w, axis=-1, keepdims=True)
    e = jnp.exp(w - m)
    p = e / jnp.sum(e, axis=-1, keepdims=True)
    o_ref[...] = lax.dot_general(
        c4t_ref[...], p, (((1,), (1,)), ((), ())),
        preferred_element_type=jnp.float32)


def _compute_coef(weight):
    out_dim = weight.shape[0]
    c4t = jnp.array(_C16, dtype=jnp.float32).T
    return pl.pallas_call(
        _coef_body,
        out_shape=jax.ShapeDtypeStruct((4, out_dim), jnp.float32),
    )(weight, c4t)


def _make_sc_kernel(batch, in_dim, out_dim):
    rows_per_tile = batch // _NW
    pass_rows = 4
    npass = rows_per_tile // pass_rows
    w = 2048
    nchunk = out_dim // w

    mesh = plsc.VectorSubcoreMesh(
        core_axis_name="c", subcore_axis_name="s",
        num_cores=_NC, num_subcores=_NS)

    @functools.partial(
        pl.kernel,
        out_type=jax.ShapeDtypeStruct((batch, out_dim), jnp.float32),
        mesh=mesh,
        compiler_params=pltpu.CompilerParams(needs_layout_passes=False),
        scratch_types=[
            pltpu.VMEM((pass_rows, in_dim), jnp.float32),
            pltpu.VMEM((2, 2, w), jnp.int32),
            pltpu.VMEM((2, 4, w), jnp.float32),
            pltpu.VMEM((2, pass_rows, w), jnp.float32),
            pltpu.SemaphoreType.DMA,
            pltpu.SemaphoreType.DMA,
            pltpu.SemaphoreType.DMA,
            pltpu.SemaphoreType.DMA,
            pltpu.SemaphoreType.DMA,
        ],
    )
    def sc_kernel(x_hbm, idx_hbm, coef_hbm, out_hbm, xbuf, ibuf, cbuf, obuf,
                  sem_x, sem_ia, sem_ib, sem_oa, sem_ob):
        sem_i = [sem_ia, sem_ib]
        sem_o = [sem_oa, sem_ob]
        wid = lax.axis_index("s") * _NC + lax.axis_index("c")
        rbase = wid * rows_per_tile

        def start_inputs(c, buf):
            hs = [pltpu.async_copy(idx_hbm.at[0, pl.ds(c * w, w)],
                                   ibuf.at[buf, 0], sem_i[buf]),
                  pltpu.async_copy(idx_hbm.at[1, pl.ds(c * w, w)],
                                   ibuf.at[buf, 1], sem_i[buf])]
            for k in range(4):
                hs.append(pltpu.async_copy(coef_hbm.at[k, pl.ds(c * w, w)],
                                           cbuf.at[buf, k], sem_i[buf]))
            return hs

        for p in range(npass):
            prow = rbase + p * pass_rows
            hx = pltpu.async_copy(x_hbm.at[pl.ds(prow, pass_rows)], xbuf,
                                  sem_x)
            in_h = {0: start_inputs(0, 0)}
            out_h = {}
            hx.wait()
            for c in range(nchunk):
                cur = c % 2
                if c + 1 < nchunk:
                    in_h[c + 1] = start_inputs(c + 1, 1 - cur)
                for h in in_h.pop(c):
                    h.wait()
                if c >= 2:
                    for h in out_h.pop(c - 2):
                        h.wait()

                def jbody(j, _):
                    o = j * _LANES
                    i0 = ibuf[cur, 0, pl.ds(o, _LANES)]
                    i1 = ibuf[cur, 1, pl.ds(o, _LANES)]
                    c0 = cbuf[cur, 0, pl.ds(o, _LANES)]
                    ca = cbuf[cur, 1, pl.ds(o, _LANES)]
                    cb = cbuf[cur, 2, pl.ds(o, _LANES)]
                    cab = cbuf[cur, 3, pl.ds(o, _LANES)]
                    for r in range(pass_rows):
                        rv = jnp.full((_LANES,), r, jnp.int32)
                        a = plsc.load_gather(xbuf, [rv, i0])
                        b = plsc.load_gather(xbuf, [rv, i1])
                        obuf[cur, r, pl.ds(o, _LANES)] = (
                            c0 + a * ca + b * cb + (a * b) * cab)
                    return 0

                lax.fori_loop(0, w // _LANES, jbody, 0)
                out_h[c] = [
                    pltpu.async_copy(obuf.at[cur, r],
                                     out_hbm.at[prow + r, pl.ds(c * w, w)],
                                     sem_o[cur])
                    for r in range(pass_rows)]
            for hs in out_h.values():
                for h in hs:
                    h.wait()

    return sc_kernel


def kernel(x, weight, indices):
    batch, in_dim = x.shape
    out_dim = weight.shape[0]
    coef = _compute_coef(weight)
    idx = indices.astype(jnp.int32)
    sc = _make_sc_kernel(batch, in_dim, out_dim)
    return sc(x, idx, coef)


# speedup vs baseline: 4.5521x; 1.7241x over previous
import functools

import jax
import jax.numpy as jnp
from jax import lax
from jax.experimental import pallas as pl
from jax.experimental.pallas import tpu as pltpu
from jax.experimental.pallas import tpu_sc as plsc

_C16 = (
    (0.0, 0.0, 0.0, 0.0),
    (0.0, 0.0, 0.0, 1.0),
    (0.0, 1.0, 0.0, -1.0),
    (0.0, 1.0, 0.0, 0.0),
    (0.0, 0.0, 1.0, -1.0),
    (0.0, 0.0, 1.0, 0.0),
    (0.0, 1.0, 1.0, -2.0),
    (0.0, 1.0, 1.0, -1.0),
    (1.0, -1.0, -1.0, 1.0),
    (1.0, -1.0, -1.0, 2.0),
    (1.0, 0.0, -1.0, 0.0),
    (1.0, 0.0, -1.0, 1.0),
    (1.0, -1.0, 0.0, 0.0),
    (1.0, -1.0, 0.0, 1.0),
    (1.0, 0.0, 0.0, -1.0),
    (1.0, 0.0, 0.0, 0.0),
)

_NC, _NS = 2, 16
_NW = _NC * _NS
_LANES = 16


def _coef_body(w_ref, c4t_ref, o_ref):
    w = w_ref[...]
    m = jnp.max(w, axis=-1, keepdims=True)
    e = jnp.exp(w - m)
    p = e / jnp.sum(e, axis=-1, keepdims=True)
    o_ref[...] = lax.dot_general(
        c4t_ref[...], p, (((1,), (1,)), ((), ())),
        preferred_element_type=jnp.float32)


def _compute_coef(weight):
    out_dim = weight.shape[0]
    c4t = jnp.array(_C16, dtype=jnp.float32).T
    return pl.pallas_call(
        _coef_body,
        out_shape=jax.ShapeDtypeStruct((4, out_dim), jnp.float32),
    )(weight, c4t)


def _make_sc_kernel(batch, in_dim, out_dim):
    rows_per_tile = batch // _NW
    pass_rows = 4
    npass = rows_per_tile // pass_rows
    w = 2048
    nchunk = out_dim // w

    mesh = plsc.VectorSubcoreMesh(
        core_axis_name="c", subcore_axis_name="s",
        num_cores=_NC, num_subcores=_NS)

    @functools.partial(
        pl.kernel,
        out_type=jax.ShapeDtypeStruct((batch, out_dim), jnp.float32),
        mesh=mesh,
        compiler_params=pltpu.CompilerParams(needs_layout_passes=False),
        scratch_types=[
            pltpu.VMEM((pass_rows, in_dim), jnp.float32),
            pltpu.VMEM((2, 2, w), jnp.int32),
            pltpu.VMEM((2, 4, w), jnp.float32),
            pltpu.VMEM((2, pass_rows, w), jnp.float32),
            pltpu.SemaphoreType.DMA,
            pltpu.SemaphoreType.DMA,
            pltpu.SemaphoreType.DMA,
            pltpu.SemaphoreType.DMA,
            pltpu.SemaphoreType.DMA,
        ],
    )
    def sc_kernel(x_hbm, idx_hbm, coef_hbm, out_hbm, xbuf, ibuf, cbuf, obuf,
                  sem_x, sem_ia, sem_ib, sem_oa, sem_ob):
        sem_i = [sem_ia, sem_ib]
        sem_o = [sem_oa, sem_ob]
        wid = lax.axis_index("s") * _NC + lax.axis_index("c")
        rbase = wid * rows_per_tile

        def start_inputs(c, buf):
            pltpu.async_copy(idx_hbm.at[0, pl.ds(c * w, w)],
                             ibuf.at[buf, 0], sem_i[buf])
            pltpu.async_copy(idx_hbm.at[1, pl.ds(c * w, w)],
                             ibuf.at[buf, 1], sem_i[buf])
            for k in range(4):
                pltpu.async_copy(coef_hbm.at[k, pl.ds(c * w, w)],
                                 cbuf.at[buf, k], sem_i[buf])

        def wait_inputs(buf):
            pltpu.make_async_copy(idx_hbm.at[0, pl.ds(0, w)],
                                  ibuf.at[buf, 0], sem_i[buf]).wait()
            pltpu.make_async_copy(idx_hbm.at[1, pl.ds(0, w)],
                                  ibuf.at[buf, 1], sem_i[buf]).wait()
            for k in range(4):
                pltpu.make_async_copy(coef_hbm.at[k, pl.ds(0, w)],
                                      cbuf.at[buf, k], sem_i[buf]).wait()

        def drain_outputs(prow, buf):
            for r in range(pass_rows):
                pltpu.make_async_copy(obuf.at[buf, r],
                                      out_hbm.at[prow + r, pl.ds(0, w)],
                                      sem_o[buf]).wait()

        def compute_chunk(c, buf, prow):
            @plsc.parallel_loop(0, w, step=_LANES, unroll=4)
            def jbody(o):
                i0 = ibuf[buf, 0, pl.ds(o, _LANES)]
                i1 = ibuf[buf, 1, pl.ds(o, _LANES)]
                c0 = cbuf[buf, 0, pl.ds(o, _LANES)]
                ca = cbuf[buf, 1, pl.ds(o, _LANES)]
                cb = cbuf[buf, 2, pl.ds(o, _LANES)]
                cab = cbuf[buf, 3, pl.ds(o, _LANES)]
                for r in range(pass_rows):
                    rv = jnp.full((_LANES,), r, jnp.int32)
                    a = plsc.load_gather(xbuf, [rv, i0])
                    b = plsc.load_gather(xbuf, [rv, i1])
                    obuf[buf, r, pl.ds(o, _LANES)] = (
                        c0 + a * ca + b * cb + (a * b) * cab)

            for r in range(pass_rows):
                pltpu.async_copy(obuf.at[buf, r],
                                 out_hbm.at[prow + r, pl.ds(c * w, w)],
                                 sem_o[buf])

        for p in range(npass):
            prow = rbase + p * pass_rows
            hx = pltpu.async_copy(x_hbm.at[pl.ds(prow, pass_rows)], xbuf,
                                  sem_x)
            start_inputs(0, 0)
            hx.wait()

            @pl.loop(0, nchunk // 2)
            def chunk_pair(t):
                c0_, c1_ = 2 * t, 2 * t + 1
                start_inputs(c1_, 1)
                wait_inputs(0)

                @pl.when(t > 0)
                def _():
                    drain_outputs(prow, 0)

                compute_chunk(c0_, 0, prow)

                @pl.when(t + 1 < nchunk // 2)
                def _():
                    start_inputs(c0_ + 2, 0)

                wait_inputs(1)

                @pl.when(t > 0)
                def _():
                    drain_outputs(prow, 1)

                compute_chunk(c1_, 1, prow)

            drain_outputs(prow, 0)
            drain_outputs(prow, 1)

    return sc_kernel


def kernel(x, weight, indices):
    batch, in_dim = x.shape
    out_dim = weight.shape[0]
    coef = _compute_coef(weight)
    idx = indices.astype(jnp.int32)
    sc = _make_sc_kernel(batch, in_dim, out_dim)
    return sc(x, idx, coef)


# speedup vs baseline: 4.8892x; 1.0741x over previous
import functools

import jax
import jax.numpy as jnp
from jax import lax
from jax.experimental import pallas as pl
from jax.experimental.pallas import tpu as pltpu
from jax.experimental.pallas import tpu_sc as plsc

_C16 = (
    (0.0, 0.0, 0.0, 0.0),
    (0.0, 0.0, 0.0, 1.0),
    (0.0, 1.0, 0.0, -1.0),
    (0.0, 1.0, 0.0, 0.0),
    (0.0, 0.0, 1.0, -1.0),
    (0.0, 0.0, 1.0, 0.0),
    (0.0, 1.0, 1.0, -2.0),
    (0.0, 1.0, 1.0, -1.0),
    (1.0, -1.0, -1.0, 1.0),
    (1.0, -1.0, -1.0, 2.0),
    (1.0, 0.0, -1.0, 0.0),
    (1.0, 0.0, -1.0, 1.0),
    (1.0, -1.0, 0.0, 0.0),
    (1.0, -1.0, 0.0, 1.0),
    (1.0, 0.0, 0.0, -1.0),
    (1.0, 0.0, 0.0, 0.0),
)

_NC, _NS = 2, 16
_NW = _NC * _NS
_LANES = 16


def _coef_body(w_ref, c4t_ref, o_ref):
    w = w_ref[...]
    m = jnp.max(w, axis=-1, keepdims=True)
    e = jnp.exp(w - m)
    p = e / jnp.sum(e, axis=-1, keepdims=True)
    o_ref[...] = lax.dot_general(
        c4t_ref[...], p, (((1,), (1,)), ((), ())),
        preferred_element_type=jnp.float32)


def _compute_coef(weight):
    out_dim = weight.shape[0]
    c4t = jnp.array(_C16, dtype=jnp.float32).T
    return pl.pallas_call(
        _coef_body,
        out_shape=jax.ShapeDtypeStruct((4, out_dim), jnp.float32),
    )(weight, c4t)


def _make_sc_kernel(batch, in_dim, out_dim):
    rows_per_tile = batch // _NW
    pass_rows = 4
    npass = rows_per_tile // pass_rows
    w = 2048
    nchunk = out_dim // w

    mesh = plsc.VectorSubcoreMesh(
        core_axis_name="c", subcore_axis_name="s",
        num_cores=_NC, num_subcores=_NS)

    @functools.partial(
        pl.kernel,
        out_type=jax.ShapeDtypeStruct((batch, out_dim), jnp.float32),
        mesh=mesh,
        compiler_params=pltpu.CompilerParams(needs_layout_passes=False),
        scratch_types=[
            pltpu.VMEM((pass_rows, in_dim), jnp.float32),
            pltpu.VMEM((2, 2, w), jnp.int32),
            pltpu.VMEM((2, 4, w), jnp.float32),
            pltpu.VMEM((2, pass_rows, w), jnp.float32),
            pltpu.VMEM_SHARED((2, out_dim), jnp.int32),
            pltpu.VMEM_SHARED((4, out_dim), jnp.float32),
            pltpu.SemaphoreType.DMA,
            pltpu.SemaphoreType.DMA,
            pltpu.SemaphoreType.DMA,
            pltpu.SemaphoreType.DMA,
            pltpu.SemaphoreType.DMA,
        ],
    )
    def sc_kernel(x_hbm, idx_hbm, coef_hbm, out_hbm, xbuf, ibuf, cbuf, obuf,
                  sh_idx, sh_coef, sem_x, sem_ia, sem_ib, sem_oa, sem_ob):
        sem_i = [sem_ia, sem_ib]
        sem_o = [sem_oa, sem_ob]
        wid = lax.axis_index("s") * _NC + lax.axis_index("c")
        rbase = wid * rows_per_tile

        @pl.when(lax.axis_index("s") == 0)
        def _():
            pltpu.sync_copy(idx_hbm, sh_idx)
            pltpu.sync_copy(coef_hbm, sh_coef)

        plsc.subcore_barrier()

        def start_inputs(c, buf):
            pltpu.async_copy(sh_idx.at[0, pl.ds(c * w, w)],
                             ibuf.at[buf, 0], sem_i[buf])
            pltpu.async_copy(sh_idx.at[1, pl.ds(c * w, w)],
                             ibuf.at[buf, 1], sem_i[buf])
            for k in range(4):
                pltpu.async_copy(sh_coef.at[k, pl.ds(c * w, w)],
                                 cbuf.at[buf, k], sem_i[buf])

        def wait_inputs(buf):
            pltpu.make_async_copy(sh_idx.at[0, pl.ds(0, w)],
                                  ibuf.at[buf, 0], sem_i[buf]).wait()
            pltpu.make_async_copy(sh_idx.at[1, pl.ds(0, w)],
                                  ibuf.at[buf, 1], sem_i[buf]).wait()
            for k in range(4):
                pltpu.make_async_copy(sh_coef.at[k, pl.ds(0, w)],
                                      cbuf.at[buf, k], sem_i[buf]).wait()

        def drain_outputs(prow, buf):
            for r in range(pass_rows):
                pltpu.make_async_copy(obuf.at[buf, r],
                                      out_hbm.at[prow + r, pl.ds(0, w)],
                                      sem_o[buf]).wait()

        def compute_chunk(c, buf, prow):
            @plsc.parallel_loop(0, w, step=_LANES, unroll=4)
            def jbody(o):
                i0 = ibuf[buf, 0, pl.ds(o, _LANES)]
                i1 = ibuf[buf, 1, pl.ds(o, _LANES)]
                c0 = cbuf[buf, 0, pl.ds(o, _LANES)]
                ca = cbuf[buf, 1, pl.ds(o, _LANES)]
                cb = cbuf[buf, 2, pl.ds(o, _LANES)]
                cab = cbuf[buf, 3, pl.ds(o, _LANES)]
                for r in range(pass_rows):
                    rv = jnp.full((_LANES,), r, jnp.int32)
                    a = plsc.load_gather(xbuf, [rv, i0])
                    b = plsc.load_gather(xbuf, [rv, i1])
                    obuf[buf, r, pl.ds(o, _LANES)] = (
                        c0 + a * ca + b * cb + (a * b) * cab)

            for r in range(pass_rows):
                pltpu.async_copy(obuf.at[buf, r],
                                 out_hbm.at[prow + r, pl.ds(c * w, w)],
                                 sem_o[buf])

        for p in range(npass):
            prow = rbase + p * pass_rows
            hx = pltpu.async_copy(x_hbm.at[pl.ds(prow, pass_rows)], xbuf,
                                  sem_x)
            start_inputs(0, 0)
            hx.wait()

            @pl.loop(0, nchunk // 2)
            def chunk_pair(t):
                c0_, c1_ = 2 * t, 2 * t + 1
                start_inputs(c1_, 1)
                wait_inputs(0)

                @pl.when(t > 0)
                def _():
                    drain_outputs(prow, 0)

                compute_chunk(c0_, 0, prow)

                @pl.when(t + 1 < nchunk // 2)
                def _():
                    start_inputs(c0_ + 2, 0)

                wait_inputs(1)

                @pl.when(t > 0)
                def _():
                    drain_outputs(prow, 1)

                compute_chunk(c1_, 1, prow)

            drain_outputs(prow, 0)
            drain_outputs(prow, 1)

    return sc_kernel


def kernel(x, weight, indices):
    batch, in_dim = x.shape
    out_dim = weight.shape[0]
    coef = _compute_coef(weight)
    idx = indices.astype(jnp.int32)
    sc = _make_sc_kernel(batch, in_dim, out_dim)
    return sc(x, idx, coef)
